# baseline (speedup 1.0000x reference)
import functools

import jax
import jax.numpy as jnp
from jax import lax
from jax.experimental import pallas as pl
from jax.experimental.pallas import tpu as pltpu
from jax.experimental.pallas import tpu_sc as plsc

N = 10000
E = 160000
F = 16

_info = plsc.get_sparse_core_info()
NC, NS = _info.num_cores, _info.num_subcores
NW = NC * NS
EPW = E // NW
CH = 125
NCHUNK = EPW // CH
MB = 1000
NMB = EPW // MB
CPM = MB // CH
NP = 10240
STRIPE = NP // NS

_mesh = plsc.VectorSubcoreMesh(core_axis_name="c", subcore_axis_name="s")
_sc_params = pltpu.CompilerParams(use_tc_tiling_on_sc=False)


@functools.partial(
    pl.kernel, mesh=_mesh, compiler_params=_sc_params,
    out_type=jax.ShapeDtypeStruct((E, F), jnp.float32),
    scratch_types=[
        pltpu.VMEM((NCHUNK, CH), jnp.int32),
        pltpu.VMEM((MB, F), jnp.float32),
        pltpu.SemaphoreType.DMA,
    ],
)
def _sc_gather(table_hbm, src_hbm, out_hbm, idx_v, rows_v, sem):
    wid = lax.axis_index("s") * NC + lax.axis_index("c")
    base = wid * EPW
    pltpu.sync_copy(src_hbm.at[wid], idx_v)

    def body(m, carry):
        handles = [
            pltpu.async_copy(table_hbm.at[idx_v.at[m * CPM + jj]],
                             rows_v.at[pl.ds(jj * CH, CH)], sem)
            for jj in range(CPM)
        ]
        for hh in handles:
            hh.wait()
        pltpu.sync_copy(rows_v, out_hbm.at[pl.ds(base + m * MB, MB)])
        return carry

    lax.fori_loop(0, NMB, body, 0)


@functools.partial(
    pl.kernel, mesh=_mesh, compiler_params=_sc_params,
    out_type=jax.ShapeDtypeStruct((NC, NP, F), jnp.float32),
    scratch_types=[
        pltpu.VMEM((NCHUNK, CH), jnp.int32),
        pltpu.VMEM((MB, F), jnp.float32),
        pltpu.VMEM((STRIPE, F), jnp.float32),
        pltpu.VMEM_SHARED((NP, F), jnp.float32),
        pltpu.SemaphoreType.DMA,
    ],
)
def _sc_scatter(msg_hbm, dst_hbm, out_hbm, idx_v, msg_v, stripe_v, acc_sh, sem):
    cid = lax.axis_index("c")
    sid = lax.axis_index("s")
    wid = sid * NC + cid

    def zbody(i, carry):
        stripe_v[i, :] = jnp.zeros((F,), jnp.float32)
        return carry

    lax.fori_loop(0, STRIPE, zbody, 0)
    pltpu.sync_copy(stripe_v, acc_sh.at[pl.ds(sid * STRIPE, STRIPE)])
    pltpu.sync_copy(dst_hbm.at[wid], idx_v)
    plsc.subcore_barrier()

    def body(m, carry):
        pltpu.async_copy(
            msg_hbm.at[pl.ds(wid * EPW + m * MB, MB)], msg_v, sem).wait()
        for jj in range(CPM):
            pltpu.sync_copy(msg_v.at[pl.ds(jj * CH, CH)],
                            acc_sh.at[idx_v.at[m * CPM + jj]], add=True)
        return carry

    lax.fori_loop(0, NMB, body, 0)
    plsc.subcore_barrier()
    pltpu.sync_copy(acc_sh.at[pl.ds(sid * STRIPE, STRIPE)], stripe_v)
    pltpu.sync_copy(stripe_v, out_hbm.at[cid, pl.ds(sid * STRIPE, STRIPE)])


@functools.partial(
    pl.kernel, mesh=_mesh, compiler_params=_sc_params,
    out_type=jax.ShapeDtypeStruct((NC, NP, F), jnp.float32),
    scratch_types=[
        pltpu.VMEM((NCHUNK, CH), jnp.int32),
        pltpu.VMEM((CH, F), jnp.float32),
        pltpu.VMEM((STRIPE, F), jnp.float32),
        pltpu.VMEM_SHARED((NP, F), jnp.float32),
    ],
)
def _sc_counts(dst_hbm, out_hbm, idx_v, ones_v, stripe_v, acc_sh):
    cid = lax.axis_index("c")
    sid = lax.axis_index("s")
    wid = sid * NC + cid

    def zbody(i, carry):
        stripe_v[i, :] = jnp.zeros((F,), jnp.float32)
        return carry

    lax.fori_loop(0, STRIPE, zbody, 0)

    def obody(i, carry):
        ones_v[i, :] = jnp.ones((F,), jnp.float32)
        return carry

    lax.fori_loop(0, CH, obody, 0)
    pltpu.sync_copy(stripe_v, acc_sh.at[pl.ds(sid * STRIPE, STRIPE)])
    pltpu.sync_copy(dst_hbm.at[wid], idx_v)
    plsc.subcore_barrier()

    def body(j, carry):
        pltpu.sync_copy(ones_v, acc_sh.at[idx_v.at[j]], add=True)
        return carry

    lax.fori_loop(0, NCHUNK, body, 0)
    plsc.subcore_barrier()
    pltpu.sync_copy(acc_sh.at[pl.ds(sid * STRIPE, STRIPE)], stripe_v)
    pltpu.sync_copy(stripe_v, out_hbm.at[cid, pl.ds(sid * STRIPE, STRIPE)])


_EB = 6400
_NB = 2000


def _msg_body(eat_ref, xjt_ref, w1t_ref, b1t_ref, w2f_ref, o_ref):
    eat = eat_ref[...]
    xjt = xjt_ref[...]
    h = jnp.maximum(
        jnp.dot(w1t_ref[...], eat, preferred_element_type=jnp.float32,
                precision=lax.Precision.HIGHEST)
        + b1t_ref[...], 0.0)
    t = jnp.dot(w2f_ref[...], xjt,
                preferred_element_type=jnp.float32,
                precision=lax.Precision.HIGHEST)
    acc = h[10:11, :] * t[160:176, :]
    for k in range(10):
        acc = acc + h[k:k + 1, :] * t[k * F:(k + 1) * F, :]
    o_ref[...] = acc


def _tc_msg(eat, xjt, W1pT, b1pT, W2f):
    return pl.pallas_call(
        _msg_body,
        grid=(E // _EB,),
        in_specs=[
            pl.BlockSpec((2, _EB), lambda i: (0, i)),
            pl.BlockSpec((F, _EB), lambda i: (0, i)),
            pl.BlockSpec((F, 2), lambda i: (0, 0)),
            pl.BlockSpec((F, 1), lambda i: (0, 0)),
            pl.BlockSpec((11 * F, F), lambda i: (0, 0)),
        ],
        out_specs=pl.BlockSpec((F, _EB), lambda i: (0, i)),
        out_shape=jax.ShapeDtypeStruct((F, E), jnp.float32),
    )(eat, xjt, W1pT, b1pT, W2f)


def _update_body(acc_ref, cnt_ref, x_ref, root_ref, bias_ref, o_ref):
    s = acc_ref[0] + acc_ref[1]
    c = cnt_ref[0] + cnt_ref[1]
    mean = s / jnp.maximum(c, 1.0)
    o_ref[...] = jnp.maximum(
        mean + jnp.dot(x_ref[...], root_ref[...],
                       preferred_element_type=jnp.float32,
                 precision=lax.Precision.HIGHEST)
        + bias_ref[...], 0.0)


def _tc_update(acc2, cnt2, x, rootp, biasp):
    return pl.pallas_call(
        _update_body,
        grid=(N // _NB,),
        in_specs=[
            pl.BlockSpec((2, _NB, F), lambda i: (0, i, 0)),
            pl.BlockSpec((2, _NB, F), lambda i: (0, i, 0)),
            pl.BlockSpec((_NB, F), lambda i: (i, 0)),
            pl.BlockSpec((F, F), lambda i: (0, 0)),
            pl.BlockSpec((1, F), lambda i: (0, 0)),
        ],
        out_specs=pl.BlockSpec((_NB, F), lambda i: (i, 0)),
        out_shape=jax.ShapeDtypeStruct((N, F), jnp.float32),
    )(acc2, cnt2, x, rootp, biasp)


def _final_body(acc_ref, cnt_ref, x_ref, root_ref, bias_ref, ow_ref, ob_ref,
                o_ref):
    s = acc_ref[0] + acc_ref[1]
    c = cnt_ref[0] + cnt_ref[1]
    mean = s / jnp.maximum(c, 1.0)
    h = jnp.maximum(
        mean + jnp.dot(x_ref[...], root_ref[...],
                       preferred_element_type=jnp.float32,
                 precision=lax.Precision.HIGHEST)
        + bias_ref[...], 0.0)
    o_ref[...] = jnp.dot(h, ow_ref[...],
                         preferred_element_type=jnp.float32,
                 precision=lax.Precision.HIGHEST) + ob_ref[...]


def _tc_final(acc2, cnt2, x, rootp, biasp, outWp, out_b):
    return pl.pallas_call(
        _final_body,
        grid=(N // _NB,),
        in_specs=[
            pl.BlockSpec((2, _NB, F), lambda i: (0, i, 0)),
            pl.BlockSpec((2, _NB, F), lambda i: (0, i, 0)),
            pl.BlockSpec((_NB, F), lambda i: (i, 0)),
            pl.BlockSpec((F, F), lambda i: (0, 0)),
            pl.BlockSpec((1, F), lambda i: (0, 0)),
            pl.BlockSpec((F, 1), lambda i: (0, 0)),
            pl.BlockSpec((1, 1), lambda i: (0, 0)),
        ],
        out_specs=pl.BlockSpec((_NB, 1), lambda i: (i, 0)),
        out_shape=jax.ShapeDtypeStruct((N, 1), jnp.float32),
    )(acc2, cnt2, x, rootp, biasp, outWp, out_b)


def _pad_layer(W1, b1, W2, b2, root, bias, cin, cout):
    W1p = jnp.pad(W1, ((0, 0), (0, F - 10)))
    b1p = jnp.pad(b1, (0, F - 10)).at[10].set(1.0).reshape(1, F)
    W2r = jnp.pad(W2.reshape(10, cin, cout),
                  ((0, 0), (0, F - cin), (0, F - cout)))
    B2r = jnp.pad(b2.reshape(cin, cout), ((0, F - cin), (0, F - cout)))
    W2s = jnp.concatenate([W2r, B2r[None]], axis=0)
    W2f = jnp.transpose(W2s, (0, 2, 1)).reshape(11 * F, F)
    rootp = jnp.pad(root, ((0, F - cin), (0, F - cout)))
    biasp = jnp.pad(bias, (0, F - cout)).reshape(1, F)
    return W1p.T, b1p.reshape(F, 1), W2f, rootp, biasp


def kernel(x, edge_index, edge_attr,
           l1_W1, l1_b1, l1_W2, l1_b2, l1_root, l1_bias,
           l2_W1, l2_b1, l2_W2, l2_b2, l2_root, l2_bias,
           l3_W1, l3_b1, l3_W2, l3_b2, l3_root, l3_bias,
           l4_W1, l4_b1, l4_W2, l4_b2, l4_root, l4_bias,
           out_W, out_b):
    src = edge_index[0].astype(jnp.int32).reshape(NW, NCHUNK, CH)
    dst = edge_index[1].astype(jnp.int32).reshape(NW, NCHUNK, CH)
    ea = edge_attr

    cnt2 = _sc_counts(dst)

    layers = [
        _pad_layer(l1_W1, l1_b1, l1_W2, l1_b2, l1_root, l1_bias, 1, F),
        _pad_layer(l2_W1, l2_b1, l2_W2, l2_b2, l2_root, l2_bias, F, F),
        _pad_layer(l3_W1, l3_b1, l3_W2, l3_b2, l3_root, l3_bias, F, F),
        _pad_layer(l4_W1, l4_b1, l4_W2, l4_b2, l4_root, l4_bias, F, 10),
    ]

    h = jnp.pad(x, ((0, 0), (0, F - 1)))
    eat = ea.T
    out = None
    for li, (W1pT, b1pT, W2f, rootp, biasp) in enumerate(layers):
        xj = _sc_gather(h, src)
        msgt = _tc_msg(eat, xj.T, W1pT, b1pT, W2f)
        acc2 = _sc_scatter(msgt.T, dst)
        if li < 3:
            h = _tc_update(acc2, cnt2, h, rootp, biasp)
        else:
            outWp = jnp.pad(out_W, ((0, F - 10), (0, 0)))
            out = _tc_final(acc2, cnt2, h, rootp, biasp, outWp,
                            out_b.reshape(1, 1))
    return out


# speedup vs baseline: 1.2177x; 1.2177x over previous
import functools

import jax
import jax.numpy as jnp
from jax import lax
from jax.experimental import pallas as pl
from jax.experimental.pallas import tpu as pltpu
from jax.experimental.pallas import tpu_sc as plsc

N = 10000
E = 160000
F = 16

_info = plsc.get_sparse_core_info()
NC, NS = _info.num_cores, _info.num_subcores
NW = NC * NS
EPW = E // NW
CH = 125
NCHUNK = EPW // CH
MB = 1000
NMB = EPW // MB
CPM = MB // CH
NP = 10240
STRIPE = NP // NS

_mesh = plsc.VectorSubcoreMesh(core_axis_name="c", subcore_axis_name="s")
_sc_params = pltpu.CompilerParams(use_tc_tiling_on_sc=False)


@functools.partial(
    pl.kernel, mesh=_mesh, compiler_params=_sc_params,
    out_type=jax.ShapeDtypeStruct((E, F), jnp.float32),
    scratch_types=[
        pltpu.VMEM((NCHUNK, CH), jnp.int32),
        pltpu.VMEM((MB, F), jnp.float32),
        pltpu.SemaphoreType.DMA,
    ],
)
def _sc_gather(table_hbm, src_hbm, out_hbm, idx_v, rows_v, sem):
    wid = lax.axis_index("s") * NC + lax.axis_index("c")
    base = wid * EPW
    pltpu.sync_copy(src_hbm.at[wid], idx_v)

    def body(m, carry):
        handles = [
            pltpu.async_copy(table_hbm.at[idx_v.at[m * CPM + jj]],
                             rows_v.at[pl.ds(jj * CH, CH)], sem)
            for jj in range(CPM)
        ]
        for hh in handles:
            hh.wait()
        pltpu.sync_copy(rows_v, out_hbm.at[pl.ds(base + m * MB, MB)])
        return carry

    lax.fori_loop(0, NMB, body, 0)


@functools.partial(
    pl.kernel, mesh=_mesh, compiler_params=_sc_params,
    out_type=jax.ShapeDtypeStruct((NC, NP, F), jnp.float32),
    scratch_types=[
        pltpu.VMEM((NCHUNK, CH), jnp.int32),
        pltpu.VMEM((MB, F), jnp.float32),
        pltpu.VMEM((STRIPE, F), jnp.float32),
        pltpu.VMEM_SHARED((NP, F), jnp.float32),
        pltpu.SemaphoreType.DMA,
    ],
)
def _sc_scatter(msg_hbm, dst_hbm, out_hbm, idx_v, msg_v, stripe_v, acc_sh, sem):
    cid = lax.axis_index("c")
    sid = lax.axis_index("s")
    wid = sid * NC + cid

    def zbody(i, carry):
        stripe_v[i, :] = jnp.zeros((F,), jnp.float32)
        return carry

    lax.fori_loop(0, STRIPE, zbody, 0)
    pltpu.sync_copy(stripe_v, acc_sh.at[pl.ds(sid * STRIPE, STRIPE)])
    pltpu.sync_copy(dst_hbm.at[wid], idx_v)
    plsc.subcore_barrier()

    def body(m, carry):
        pltpu.async_copy(
            msg_hbm.at[pl.ds(wid * EPW + m * MB, MB)], msg_v, sem).wait()
        for jj in range(CPM):
            pltpu.sync_copy(msg_v.at[pl.ds(jj * CH, CH)],
                            acc_sh.at[idx_v.at[m * CPM + jj]], add=True)
        return carry

    lax.fori_loop(0, NMB, body, 0)
    plsc.subcore_barrier()
    pltpu.sync_copy(acc_sh.at[pl.ds(sid * STRIPE, STRIPE)], stripe_v)
    pltpu.sync_copy(stripe_v, out_hbm.at[cid, pl.ds(sid * STRIPE, STRIPE)])


@functools.partial(
    pl.kernel, mesh=_mesh, compiler_params=_sc_params,
    out_type=jax.ShapeDtypeStruct((NC, NP, F), jnp.float32),
    scratch_types=[
        pltpu.VMEM((NCHUNK, CH), jnp.int32),
        pltpu.VMEM((CH, F), jnp.float32),
        pltpu.VMEM((STRIPE, F), jnp.float32),
        pltpu.VMEM_SHARED((NP, F), jnp.float32),
    ],
)
def _sc_counts(dst_hbm, out_hbm, idx_v, ones_v, stripe_v, acc_sh):
    cid = lax.axis_index("c")
    sid = lax.axis_index("s")
    wid = sid * NC + cid

    def zbody(i, carry):
        stripe_v[i, :] = jnp.zeros((F,), jnp.float32)
        return carry

    lax.fori_loop(0, STRIPE, zbody, 0)

    def obody(i, carry):
        ones_v[i, :] = jnp.ones((F,), jnp.float32)
        return carry

    lax.fori_loop(0, CH, obody, 0)
    pltpu.sync_copy(stripe_v, acc_sh.at[pl.ds(sid * STRIPE, STRIPE)])
    pltpu.sync_copy(dst_hbm.at[wid], idx_v)
    plsc.subcore_barrier()

    def body(j, carry):
        pltpu.sync_copy(ones_v, acc_sh.at[idx_v.at[j]], add=True)
        return carry

    lax.fori_loop(0, NCHUNK, body, 0)
    plsc.subcore_barrier()
    pltpu.sync_copy(acc_sh.at[pl.ds(sid * STRIPE, STRIPE)], stripe_v)
    pltpu.sync_copy(stripe_v, out_hbm.at[cid, pl.ds(sid * STRIPE, STRIPE)])


_EB = 6400
_NB = 2000


def _msg_body(eat_ref, xj_ref, w1t_ref, b1t_ref, w2f_ref, o_ref):
    eat = eat_ref[...]
    xjt = xj_ref[...].T
    h = jnp.maximum(
        jnp.dot(w1t_ref[...], eat, preferred_element_type=jnp.float32,
                precision=lax.Precision.HIGHEST)
        + b1t_ref[...], 0.0)
    t = jnp.dot(w2f_ref[...], xjt,
                preferred_element_type=jnp.float32,
                precision=lax.Precision.HIGHEST)
    acc = h[10:11, :] * t[160:176, :]
    for k in range(10):
        acc = acc + h[k:k + 1, :] * t[k * F:(k + 1) * F, :]
    o_ref[...] = acc.T


def _tc_msg(eat, xj, W1pT, b1pT, W2f):
    return pl.pallas_call(
        _msg_body,
        grid=(E // _EB,),
        in_specs=[
            pl.BlockSpec((2, _EB), lambda i: (0, i)),
            pl.BlockSpec((_EB, F), lambda i: (i, 0)),
            pl.BlockSpec((F, 2), lambda i: (0, 0)),
            pl.BlockSpec((F, 1), lambda i: (0, 0)),
            pl.BlockSpec((11 * F, F), lambda i: (0, 0)),
        ],
        out_specs=pl.BlockSpec((_EB, F), lambda i: (i, 0)),
        out_shape=jax.ShapeDtypeStruct((E, F), jnp.float32),
    )(eat, xj, W1pT, b1pT, W2f)


def _update_body(acc_ref, cnt_ref, x_ref, root_ref, bias_ref, o_ref):
    s = acc_ref[0] + acc_ref[1]
    c = cnt_ref[0] + cnt_ref[1]
    mean = s / jnp.maximum(c, 1.0)
    o_ref[...] = jnp.maximum(
        mean + jnp.dot(x_ref[...], root_ref[...],
                       preferred_element_type=jnp.float32,
                 precision=lax.Precision.HIGHEST)
        + bias_ref[...], 0.0)


def _tc_update(acc2, cnt2, x, rootp, biasp):
    return pl.pallas_call(
        _update_body,
        grid=(N // _NB,),
        in_specs=[
            pl.BlockSpec((2, _NB, F), lambda i: (0, i, 0)),
            pl.BlockSpec((2, _NB, F), lambda i: (0, i, 0)),
            pl.BlockSpec((_NB, F), lambda i: (i, 0)),
            pl.BlockSpec((F, F), lambda i: (0, 0)),
            pl.BlockSpec((1, F), lambda i: (0, 0)),
        ],
        out_specs=pl.BlockSpec((_NB, F), lambda i: (i, 0)),
        out_shape=jax.ShapeDtypeStruct((N, F), jnp.float32),
    )(acc2, cnt2, x, rootp, biasp)


def _final_body(acc_ref, cnt_ref, x_ref, root_ref, bias_ref, ow_ref, ob_ref,
                o_ref):
    s = acc_ref[0] + acc_ref[1]
    c = cnt_ref[0] + cnt_ref[1]
    mean = s / jnp.maximum(c, 1.0)
    h = jnp.maximum(
        mean + jnp.dot(x_ref[...], root_ref[...],
                       preferred_element_type=jnp.float32,
                 precision=lax.Precision.HIGHEST)
        + bias_ref[...], 0.0)
    o_ref[...] = jnp.dot(h, ow_ref[...],
                         preferred_element_type=jnp.float32,
                 precision=lax.Precision.HIGHEST) + ob_ref[...]


def _tc_final(acc2, cnt2, x, rootp, biasp, outWp, out_b):
    return pl.pallas_call(
        _final_body,
        grid=(N // _NB,),
        in_specs=[
            pl.BlockSpec((2, _NB, F), lambda i: (0, i, 0)),
            pl.BlockSpec((2, _NB, F), lambda i: (0, i, 0)),
            pl.BlockSpec((_NB, F), lambda i: (i, 0)),
            pl.BlockSpec((F, F), lambda i: (0, 0)),
            pl.BlockSpec((1, F), lambda i: (0, 0)),
            pl.BlockSpec((F, 1), lambda i: (0, 0)),
            pl.BlockSpec((1, 1), lambda i: (0, 0)),
        ],
        out_specs=pl.BlockSpec((_NB, 1), lambda i: (i, 0)),
        out_shape=jax.ShapeDtypeStruct((N, 1), jnp.float32),
    )(acc2, cnt2, x, rootp, biasp, outWp, out_b)


def _pad_layer(W1, b1, W2, b2, root, bias, cin, cout):
    W1p = jnp.pad(W1, ((0, 0), (0, F - 10)))
    b1p = jnp.pad(b1, (0, F - 10)).at[10].set(1.0).reshape(1, F)
    W2r = jnp.pad(W2.reshape(10, cin, cout),
                  ((0, 0), (0, F - cin), (0, F - cout)))
    B2r = jnp.pad(b2.reshape(cin, cout), ((0, F - cin), (0, F - cout)))
    W2s = jnp.concatenate([W2r, B2r[None]], axis=0)
    W2f = jnp.transpose(W2s, (0, 2, 1)).reshape(11 * F, F)
    rootp = jnp.pad(root, ((0, F - cin), (0, F - cout)))
    biasp = jnp.pad(bias, (0, F - cout)).reshape(1, F)
    return W1p.T, b1p.reshape(F, 1), W2f, rootp, biasp


def kernel(x, edge_index, edge_attr,
           l1_W1, l1_b1, l1_W2, l1_b2, l1_root, l1_bias,
           l2_W1, l2_b1, l2_W2, l2_b2, l2_root, l2_bias,
           l3_W1, l3_b1, l3_W2, l3_b2, l3_root, l3_bias,
           l4_W1, l4_b1, l4_W2, l4_b2, l4_root, l4_bias,
           out_W, out_b):
    src = edge_index[0].astype(jnp.int32).reshape(NW, NCHUNK, CH)
    dst = edge_index[1].astype(jnp.int32).reshape(NW, NCHUNK, CH)
    ea = edge_attr

    cnt2 = _sc_counts(dst)

    layers = [
        _pad_layer(l1_W1, l1_b1, l1_W2, l1_b2, l1_root, l1_bias, 1, F),
        _pad_layer(l2_W1, l2_b1, l2_W2, l2_b2, l2_root, l2_bias, F, F),
        _pad_layer(l3_W1, l3_b1, l3_W2, l3_b2, l3_root, l3_bias, F, F),
        _pad_layer(l4_W1, l4_b1, l4_W2, l4_b2, l4_root, l4_bias, F, 10),
    ]

    h = jnp.pad(x, ((0, 0), (0, F - 1)))
    eat = ea.T
    out = None
    for li, (W1pT, b1pT, W2f, rootp, biasp) in enumerate(layers):
        xj = _sc_gather(h, src)
        msg = _tc_msg(eat, xj, W1pT, b1pT, W2f)
        acc2 = _sc_scatter(msg, dst)
        if li < 3:
            h = _tc_update(acc2, cnt2, h, rootp, biasp)
        else:
            outWp = jnp.pad(out_W, ((0, F - 10), (0, 0)))
            out = _tc_final(acc2, cnt2, h, rootp, biasp, outWp,
                            out_b.reshape(1, 1))
    return out


# speedup vs baseline: 1.6385x; 1.3456x over previous
import functools

import jax
import jax.numpy as jnp
from jax import lax
from jax.experimental import pallas as pl
from jax.experimental.pallas import tpu as pltpu
from jax.experimental.pallas import tpu_sc as plsc

N = 10000
E = 160000
F = 16

_info = plsc.get_sparse_core_info()
NC, NS = _info.num_cores, _info.num_subcores
NW = NC * NS
EPW = E // NW
CH = 128
EPWP = 5120
EP = NW * EPWP
NCHUNK = EPWP // CH
MB = 1024
NMB = EPWP // MB
CPM = MB // CH
NP = 10240
STRIPE = NP // NS

_mesh = plsc.VectorSubcoreMesh(core_axis_name="c", subcore_axis_name="s")
_sc_params = pltpu.CompilerParams(use_tc_tiling_on_sc=False)


@functools.partial(
    pl.kernel, mesh=_mesh, compiler_params=_sc_params,
    out_type=jax.ShapeDtypeStruct((NW, EPWP, F), jnp.float32),
    scratch_types=[
        pltpu.VMEM((NCHUNK, CH), jnp.int32),
        pltpu.VMEM((MB, F), jnp.float32),
        pltpu.SemaphoreType.DMA,
    ],
)
def _sc_gather(table_hbm, src_hbm, out_hbm, idx_v, rows_v, sem):
    wid = lax.axis_index("s") * NC + lax.axis_index("c")
    pltpu.sync_copy(src_hbm.at[wid], idx_v)

    def body(m, carry):
        handles = [
            pltpu.async_copy(table_hbm.at[idx_v.at[m * CPM + jj]],
                             rows_v.at[pl.ds(jj * CH, CH)], sem)
            for jj in range(CPM)
        ]
        for hh in handles:
            hh.wait()
        pltpu.sync_copy(rows_v, out_hbm.at[wid, pl.ds(m * MB, MB)])
        return carry

    lax.fori_loop(0, NMB, body, 0)


@functools.partial(
    pl.kernel, mesh=_mesh, compiler_params=_sc_params,
    out_type=jax.ShapeDtypeStruct((NC, NP, F), jnp.float32),
    scratch_types=[
        pltpu.VMEM((NCHUNK, CH), jnp.int32),
        pltpu.VMEM((MB, F), jnp.float32),
        pltpu.VMEM((STRIPE, F), jnp.float32),
        pltpu.VMEM_SHARED((NP, F), jnp.float32),
        pltpu.SemaphoreType.DMA,
    ],
)
def _sc_scatter(msg_hbm, dst_hbm, out_hbm, idx_v, msg_v, stripe_v, acc_sh, sem):
    cid = lax.axis_index("c")
    sid = lax.axis_index("s")
    wid = sid * NC + cid

    def zbody(i, carry):
        stripe_v[i, :] = jnp.zeros((F,), jnp.float32)
        return carry

    lax.fori_loop(0, STRIPE, zbody, 0)
    pltpu.sync_copy(stripe_v, acc_sh.at[pl.ds(sid * STRIPE, STRIPE)])
    pltpu.sync_copy(dst_hbm.at[wid], idx_v)
    plsc.subcore_barrier()

    def body(m, carry):
        pltpu.async_copy(
            msg_hbm.at[wid, pl.ds(m * MB, MB)], msg_v, sem).wait()
        for jj in range(CPM):
            pltpu.sync_copy(msg_v.at[pl.ds(jj * CH, CH)],
                            acc_sh.at[idx_v.at[m * CPM + jj]], add=True)
        return carry

    lax.fori_loop(0, NMB, body, 0)
    plsc.subcore_barrier()
    pltpu.sync_copy(acc_sh.at[pl.ds(sid * STRIPE, STRIPE)], stripe_v)
    pltpu.sync_copy(stripe_v, out_hbm.at[cid, pl.ds(sid * STRIPE, STRIPE)])


@functools.partial(
    pl.kernel, mesh=_mesh, compiler_params=_sc_params,
    out_type=jax.ShapeDtypeStruct((NC, NP, F), jnp.float32),
    scratch_types=[
        pltpu.VMEM((NCHUNK, CH), jnp.int32),
        pltpu.VMEM((CH, F), jnp.float32),
        pltpu.VMEM((STRIPE, F), jnp.float32),
        pltpu.VMEM_SHARED((NP, F), jnp.float32),
    ],
)
def _sc_counts(dst_hbm, out_hbm, idx_v, ones_v, stripe_v, acc_sh):
    cid = lax.axis_index("c")
    sid = lax.axis_index("s")
    wid = sid * NC + cid

    def zbody(i, carry):
        stripe_v[i, :] = jnp.zeros((F,), jnp.float32)
        return carry

    lax.fori_loop(0, STRIPE, zbody, 0)

    def obody(i, carry):
        ones_v[i, :] = jnp.ones((F,), jnp.float32)
        return carry

    lax.fori_loop(0, CH, obody, 0)
    pltpu.sync_copy(stripe_v, acc_sh.at[pl.ds(sid * STRIPE, STRIPE)])
    pltpu.sync_copy(dst_hbm.at[wid], idx_v)
    plsc.subcore_barrier()

    def body(j, carry):
        pltpu.sync_copy(ones_v, acc_sh.at[idx_v.at[j]], add=True)
        return carry

    lax.fori_loop(0, NCHUNK, body, 0)
    plsc.subcore_barrier()
    pltpu.sync_copy(acc_sh.at[pl.ds(sid * STRIPE, STRIPE)], stripe_v)
    pltpu.sync_copy(stripe_v, out_hbm.at[cid, pl.ds(sid * STRIPE, STRIPE)])


_EB = 8192
_NB = 2000


def _msg_body(eat_ref, xj_ref, w1t_ref, b1t_ref, w2f_ref, o_ref):
    xp = xj_ref[...].T
    h = jnp.maximum(
        jnp.dot(w1t_ref[...], eat_ref[...], preferred_element_type=jnp.float32,
                precision=lax.Precision.HIGHEST)
        + b1t_ref[...], 0.0)
    w2f = w2f_ref[...]
    g = _EB // 8
    cols = []
    for j in range(8):
        xjt = xp[F * j:F * (j + 1), :]
        hj = h[:, g * j:g * (j + 1)]
        t = jnp.dot(w2f, xjt, preferred_element_type=jnp.float32,
                    precision=lax.Precision.HIGHEST)
        acc = hj[10:11, :] * t[160:176, :]
        for k in range(10):
            acc = acc + hj[k:k + 1, :] * t[k * F:(k + 1) * F, :]
        cols.append(acc.T)
    o_ref[...] = jnp.concatenate(cols, axis=1)


def _tc_msg(eat, xj, W1pT, b1pT, W2f):
    return pl.pallas_call(
        _msg_body,
        grid=(EP // _EB,),
        in_specs=[
            pl.BlockSpec((2, _EB), lambda i: (0, i)),
            pl.BlockSpec((_EB // 8, 128), lambda i: (i, 0)),
            pl.BlockSpec((F, 2), lambda i: (0, 0)),
            pl.BlockSpec((F, 1), lambda i: (0, 0)),
            pl.BlockSpec((11 * F, F), lambda i: (0, 0)),
        ],
        out_specs=pl.BlockSpec((_EB // 8, 128), lambda i: (i, 0)),
        out_shape=jax.ShapeDtypeStruct((EP // 8, 128), jnp.float32),
    )(eat, xj, W1pT, b1pT, W2f)


def _update_body(acc_ref, cnt_ref, x_ref, root_ref, bias_ref, o_ref):
    s = acc_ref[0] + acc_ref[1]
    c = cnt_ref[0] + cnt_ref[1]
    mean = s / jnp.maximum(c, 1.0)
    o_ref[...] = jnp.maximum(
        mean + jnp.dot(x_ref[...], root_ref[...],
                       preferred_element_type=jnp.float32,
                 precision=lax.Precision.HIGHEST)
        + bias_ref[...], 0.0)


def _tc_update(acc2, cnt2, x, rootp, biasp):
    return pl.pallas_call(
        _update_body,
        grid=(N // _NB,),
        in_specs=[
            pl.BlockSpec((2, _NB, F), lambda i: (0, i, 0)),
            pl.BlockSpec((2, _NB, F), lambda i: (0, i, 0)),
            pl.BlockSpec((_NB, F), lambda i: (i, 0)),
            pl.BlockSpec((F, F), lambda i: (0, 0)),
            pl.BlockSpec((1, F), lambda i: (0, 0)),
        ],
        out_specs=pl.BlockSpec((_NB, F), lambda i: (i, 0)),
        out_shape=jax.ShapeDtypeStruct((N, F), jnp.float32),
    )(acc2, cnt2, x, rootp, biasp)


def _final_body(acc_ref, cnt_ref, x_ref, root_ref, bias_ref, ow_ref, ob_ref,
                o_ref):
    s = acc_ref[0] + acc_ref[1]
    c = cnt_ref[0] + cnt_ref[1]
    mean = s / jnp.maximum(c, 1.0)
    h = jnp.maximum(
        mean + jnp.dot(x_ref[...], root_ref[...],
                       preferred_element_type=jnp.float32,
                 precision=lax.Precision.HIGHEST)
        + bias_ref[...], 0.0)
    o_ref[...] = jnp.dot(h, ow_ref[...],
                         preferred_element_type=jnp.float32,
                 precision=lax.Precision.HIGHEST) + ob_ref[...]


def _tc_final(acc2, cnt2, x, rootp, biasp, outWp, out_b):
    return pl.pallas_call(
        _final_body,
        grid=(N // _NB,),
        in_specs=[
            pl.BlockSpec((2, _NB, F), lambda i: (0, i, 0)),
            pl.BlockSpec((2, _NB, F), lambda i: (0, i, 0)),
            pl.BlockSpec((_NB, F), lambda i: (i, 0)),
            pl.BlockSpec((F, F), lambda i: (0, 0)),
            pl.BlockSpec((1, F), lambda i: (0, 0)),
            pl.BlockSpec((F, 1), lambda i: (0, 0)),
            pl.BlockSpec((1, 1), lambda i: (0, 0)),
        ],
        out_specs=pl.BlockSpec((_NB, 1), lambda i: (i, 0)),
        out_shape=jax.ShapeDtypeStruct((N, 1), jnp.float32),
    )(acc2, cnt2, x, rootp, biasp, outWp, out_b)


def _pad_layer(W1, b1, W2, b2, root, bias, cin, cout):
    W1p = jnp.pad(W1, ((0, 0), (0, F - 10)))
    b1p = jnp.pad(b1, (0, F - 10)).at[10].set(1.0).reshape(1, F)
    W2r = jnp.pad(W2.reshape(10, cin, cout),
                  ((0, 0), (0, F - cin), (0, F - cout)))
    B2r = jnp.pad(b2.reshape(cin, cout), ((0, F - cin), (0, F - cout)))
    W2s = jnp.concatenate([W2r, B2r[None]], axis=0)
    W2f = jnp.transpose(W2s, (0, 2, 1)).reshape(11 * F, F)
    rootp = jnp.pad(root, ((0, F - cin), (0, F - cout)))
    biasp = jnp.pad(bias, (0, F - cout)).reshape(1, F)
    return W1p.T, b1p.reshape(F, 1), W2f, rootp, biasp


def kernel(x, edge_index, edge_attr,
           l1_W1, l1_b1, l1_W2, l1_b2, l1_root, l1_bias,
           l2_W1, l2_b1, l2_W2, l2_b2, l2_root, l2_bias,
           l3_W1, l3_b1, l3_W2, l3_b2, l3_root, l3_bias,
           l4_W1, l4_b1, l4_W2, l4_b2, l4_root, l4_bias,
           out_W, out_b):
    def pad_idx(a, fill):
        a = a.astype(jnp.int32).reshape(NW, EPW)
        a = jnp.pad(a, ((0, 0), (0, EPWP - EPW)), constant_values=fill)
        return a.reshape(NW, NCHUNK, CH)

    src = pad_idx(edge_index[0], 0)
    dst = pad_idx(edge_index[1], N)
    eat = jnp.pad(edge_attr.T.reshape(2, NW, EPW),
                  ((0, 0), (0, 0), (0, EPWP - EPW))).reshape(2, EP)
    eat = eat.reshape(2, EP // _EB, _EB // 8, 8)
    eat = eat.transpose(0, 1, 3, 2).reshape(2, EP)

    cnt2 = _sc_counts(dst)

    layers = [
        _pad_layer(l1_W1, l1_b1, l1_W2, l1_b2, l1_root, l1_bias, 1, F),
        _pad_layer(l2_W1, l2_b1, l2_W2, l2_b2, l2_root, l2_bias, F, F),
        _pad_layer(l3_W1, l3_b1, l3_W2, l3_b2, l3_root, l3_bias, F, F),
        _pad_layer(l4_W1, l4_b1, l4_W2, l4_b2, l4_root, l4_bias, F, 10),
    ]

    h = jnp.pad(x, ((0, 0), (0, F - 1)))
    out = None
    for li, (W1pT, b1pT, W2f, rootp, biasp) in enumerate(layers):
        xj = _sc_gather(h, src)
        msg = _tc_msg(eat, xj.reshape(EP // 8, 128), W1pT, b1pT, W2f)
        acc2 = _sc_scatter(msg.reshape(NW, EPWP, F), dst)
        if li < 3:
            h = _tc_update(acc2, cnt2, h, rootp, biasp)
        else:
            outWp = jnp.pad(out_W, ((0, F - 10), (0, 0)))
            out = _tc_final(acc2, cnt2, h, rootp, biasp, outWp,
                            out_b.reshape(1, 1))
    return out


# speedup vs baseline: 2.1335x; 1.3021x over previous
import functools

import jax
import jax.numpy as jnp
from jax import lax
from jax.experimental import pallas as pl
from jax.experimental.pallas import tpu as pltpu
from jax.experimental.pallas import tpu_sc as plsc

N = 10000
E = 160000
F = 16

_info = plsc.get_sparse_core_info()
NC, NS = _info.num_cores, _info.num_subcores
NW = NC * NS
EPW = E // NW
CH = 128
EPWP = 5120
EP = NW * EPWP
NCHUNK = EPWP // CH
MB = 1024
NMB = EPWP // MB
CPM = MB // CH
NP = 10240
STRIPE = NP // NS

_mesh = plsc.VectorSubcoreMesh(core_axis_name="c", subcore_axis_name="s")
_sc_params = pltpu.CompilerParams(use_tc_tiling_on_sc=False)


@functools.partial(
    pl.kernel, mesh=_mesh, compiler_params=_sc_params,
    out_type=jax.ShapeDtypeStruct((NW, EPWP, F), jnp.float32),
    scratch_types=[
        pltpu.VMEM((NCHUNK, CH), jnp.int32),
        pltpu.VMEM((MB, F), jnp.float32),
        pltpu.SemaphoreType.DMA,
    ],
)
def _sc_gather(table_hbm, src_hbm, out_hbm, idx_v, rows_v, sem):
    wid = lax.axis_index("s") * NC + lax.axis_index("c")
    pltpu.sync_copy(src_hbm.at[wid], idx_v)

    def body(m, carry):
        handles = [
            pltpu.async_copy(table_hbm.at[idx_v.at[m * CPM + jj]],
                             rows_v.at[pl.ds(jj * CH, CH)], sem)
            for jj in range(CPM)
        ]
        for hh in handles:
            hh.wait()
        pltpu.sync_copy(rows_v, out_hbm.at[wid, pl.ds(m * MB, MB)])
        return carry

    lax.fori_loop(0, NMB, body, 0)


@functools.partial(
    pl.kernel, mesh=_mesh, compiler_params=_sc_params,
    out_type=jax.ShapeDtypeStruct((NC, NP, F), jnp.float32),
    scratch_types=[
        pltpu.VMEM((NCHUNK, CH), jnp.int32),
        pltpu.VMEM((MB, F), jnp.float32),
        pltpu.VMEM((STRIPE, F), jnp.float32),
        pltpu.VMEM_SHARED((NP, F), jnp.float32),
        pltpu.SemaphoreType.DMA,
    ],
)
def _sc_scatter(msg_hbm, dst_hbm, out_hbm, idx_v, msg_v, stripe_v, acc_sh, sem):
    cid = lax.axis_index("c")
    sid = lax.axis_index("s")
    wid = sid * NC + cid

    def zbody(i, carry):
        stripe_v[i, :] = jnp.zeros((F,), jnp.float32)
        return carry

    lax.fori_loop(0, STRIPE, zbody, 0)
    pltpu.sync_copy(stripe_v, acc_sh.at[pl.ds(sid * STRIPE, STRIPE)])
    pltpu.sync_copy(dst_hbm.at[wid], idx_v)
    plsc.subcore_barrier()

    def body(m, carry):
        pltpu.async_copy(
            msg_hbm.at[wid, pl.ds(m * MB, MB)], msg_v, sem).wait()
        for jj in range(CPM):
            pltpu.sync_copy(msg_v.at[pl.ds(jj * CH, CH)],
                            acc_sh.at[idx_v.at[m * CPM + jj]], add=True)
        return carry

    lax.fori_loop(0, NMB, body, 0)
    plsc.subcore_barrier()
    pltpu.sync_copy(acc_sh.at[pl.ds(sid * STRIPE, STRIPE)], stripe_v)
    pltpu.sync_copy(stripe_v, out_hbm.at[cid, pl.ds(sid * STRIPE, STRIPE)])


@functools.partial(
    pl.kernel, mesh=_mesh, compiler_params=_sc_params,
    out_type=jax.ShapeDtypeStruct((NC, NP, F), jnp.float32),
    scratch_types=[
        pltpu.VMEM((NCHUNK, CH), jnp.int32),
        pltpu.VMEM((CH, F), jnp.float32),
        pltpu.VMEM((STRIPE, F), jnp.float32),
        pltpu.VMEM_SHARED((NP, F), jnp.float32),
    ],
)
def _sc_counts(dst_hbm, out_hbm, idx_v, ones_v, stripe_v, acc_sh):
    cid = lax.axis_index("c")
    sid = lax.axis_index("s")
    wid = sid * NC + cid

    def zbody(i, carry):
        stripe_v[i, :] = jnp.zeros((F,), jnp.float32)
        return carry

    lax.fori_loop(0, STRIPE, zbody, 0)

    def obody(i, carry):
        ones_v[i, :] = jnp.ones((F,), jnp.float32)
        return carry

    lax.fori_loop(0, CH, obody, 0)
    pltpu.sync_copy(stripe_v, acc_sh.at[pl.ds(sid * STRIPE, STRIPE)])
    pltpu.sync_copy(dst_hbm.at[wid], idx_v)
    plsc.subcore_barrier()

    def body(j, carry):
        pltpu.sync_copy(ones_v, acc_sh.at[idx_v.at[j]], add=True)
        return carry

    lax.fori_loop(0, NCHUNK, body, 0)
    plsc.subcore_barrier()
    pltpu.sync_copy(acc_sh.at[pl.ds(sid * STRIPE, STRIPE)], stripe_v)
    pltpu.sync_copy(stripe_v, out_hbm.at[cid, pl.ds(sid * STRIPE, STRIPE)])


_EB = 16384


def plt_bitcast(x):
    return lax.bitcast_convert_type(x, jnp.int32)


def plt_bitcast_f(x):
    return lax.bitcast_convert_type(x, jnp.float32)
_NB = 2000


def _msg_body(eat_ref, xj_ref, w1t_ref, b1t_ref, wh_ref, wl_ref, o_ref):
    xp = xj_ref[...].T
    eat = eat_ref[...]
    w1t = w1t_ref[...]
    h = jnp.maximum(
        w1t[:, 0:1] * eat[0:1, :] + w1t[:, 1:2] * eat[1:2, :]
        + b1t_ref[...], 0.0)
    wh = wh_ref[...]
    wl = wl_ref[...]
    g = _EB // 8
    cols = []
    for j in range(8):
        xjt = xp[F * j:F * (j + 1), :]
        hj = h[:, g * j:g * (j + 1)]
        xb = plt_bitcast(xjt)
        xh = plt_bitcast_f(xb & jnp.int32(-65536))
        xl = xjt - xh
        t = (jnp.dot(wh, xh.astype(jnp.bfloat16),
                     preferred_element_type=jnp.float32)
             + jnp.dot(wh, xl.astype(jnp.bfloat16),
                       preferred_element_type=jnp.float32)
             + jnp.dot(wl, xh.astype(jnp.bfloat16),
                       preferred_element_type=jnp.float32))
        acc = hj[10:11, :] * t[160:176, :]
        for k in range(10):
            acc = acc + hj[k:k + 1, :] * t[k * F:(k + 1) * F, :]
        cols.append(acc.T)
    o_ref[...] = jnp.concatenate(cols, axis=1)


def _tc_msg(eat, xj, W1pT, b1pT, W2fh, W2fl):
    return pl.pallas_call(
        _msg_body,
        grid=(EP // _EB,),
        in_specs=[
            pl.BlockSpec((2, _EB), lambda i: (0, i)),
            pl.BlockSpec((_EB // 8, 128), lambda i: (i, 0)),
            pl.BlockSpec((F, 2), lambda i: (0, 0)),
            pl.BlockSpec((F, 1), lambda i: (0, 0)),
            pl.BlockSpec((11 * F, F), lambda i: (0, 0)),
            pl.BlockSpec((11 * F, F), lambda i: (0, 0)),
        ],
        out_specs=pl.BlockSpec((_EB // 8, 128), lambda i: (i, 0)),
        out_shape=jax.ShapeDtypeStruct((EP // 8, 128), jnp.float32),
    )(eat, xj, W1pT, b1pT, W2fh, W2fl)


def _update_body(acc_ref, cnt_ref, x_ref, root_ref, bias_ref, o_ref):
    s = acc_ref[0] + acc_ref[1]
    c = cnt_ref[0] + cnt_ref[1]
    mean = s / jnp.maximum(c, 1.0)
    o_ref[...] = jnp.maximum(
        mean + jnp.dot(x_ref[...], root_ref[...],
                       preferred_element_type=jnp.float32,
                 precision=lax.Precision.HIGHEST)
        + bias_ref[...], 0.0)


def _tc_update(acc2, cnt2, x, rootp, biasp):
    return pl.pallas_call(
        _update_body,
        grid=(N // _NB,),
        in_specs=[
            pl.BlockSpec((2, _NB, F), lambda i: (0, i, 0)),
            pl.BlockSpec((2, _NB, F), lambda i: (0, i, 0)),
            pl.BlockSpec((_NB, F), lambda i: (i, 0)),
            pl.BlockSpec((F, F), lambda i: (0, 0)),
            pl.BlockSpec((1, F), lambda i: (0, 0)),
        ],
        out_specs=pl.BlockSpec((_NB, F), lambda i: (i, 0)),
        out_shape=jax.ShapeDtypeStruct((N, F), jnp.float32),
    )(acc2, cnt2, x, rootp, biasp)


def _final_body(acc_ref, cnt_ref, x_ref, root_ref, bias_ref, ow_ref, ob_ref,
                o_ref):
    s = acc_ref[0] + acc_ref[1]
    c = cnt_ref[0] + cnt_ref[1]
    mean = s / jnp.maximum(c, 1.0)
    h = jnp.maximum(
        mean + jnp.dot(x_ref[...], root_ref[...],
                       preferred_element_type=jnp.float32,
                 precision=lax.Precision.HIGHEST)
        + bias_ref[...], 0.0)
    o_ref[...] = jnp.dot(h, ow_ref[...],
                         preferred_element_type=jnp.float32,
                 precision=lax.Precision.HIGHEST) + ob_ref[...]


def _tc_final(acc2, cnt2, x, rootp, biasp, outWp, out_b):
    return pl.pallas_call(
        _final_body,
        grid=(N // _NB,),
        in_specs=[
            pl.BlockSpec((2, _NB, F), lambda i: (0, i, 0)),
            pl.BlockSpec((2, _NB, F), lambda i: (0, i, 0)),
            pl.BlockSpec((_NB, F), lambda i: (i, 0)),
            pl.BlockSpec((F, F), lambda i: (0, 0)),
            pl.BlockSpec((1, F), lambda i: (0, 0)),
            pl.BlockSpec((F, 1), lambda i: (0, 0)),
            pl.BlockSpec((1, 1), lambda i: (0, 0)),
        ],
        out_specs=pl.BlockSpec((_NB, 1), lambda i: (i, 0)),
        out_shape=jax.ShapeDtypeStruct((N, 1), jnp.float32),
    )(acc2, cnt2, x, rootp, biasp, outWp, out_b)


def _pad_layer(W1, b1, W2, b2, root, bias, cin, cout):
    W1p = jnp.pad(W1, ((0, 0), (0, F - 10)))
    b1p = jnp.pad(b1, (0, F - 10)).at[10].set(1.0).reshape(1, F)
    W2r = jnp.pad(W2.reshape(10, cin, cout),
                  ((0, 0), (0, F - cin), (0, F - cout)))
    B2r = jnp.pad(b2.reshape(cin, cout), ((0, F - cin), (0, F - cout)))
    W2s = jnp.concatenate([W2r, B2r[None]], axis=0)
    W2f = jnp.transpose(W2s, (0, 2, 1)).reshape(11 * F, F)
    W2fi = jax.lax.bitcast_convert_type(W2f, jnp.int32)
    W2fh = jax.lax.bitcast_convert_type(W2fi & jnp.int32(-65536), jnp.float32)
    W2fl = (W2f - W2fh).astype(jnp.bfloat16)
    W2fh = W2fh.astype(jnp.bfloat16)
    rootp = jnp.pad(root, ((0, F - cin), (0, F - cout)))
    biasp = jnp.pad(bias, (0, F - cout)).reshape(1, F)
    return W1p.T, b1p.reshape(F, 1), W2fh, W2fl, rootp, biasp


def kernel(x, edge_index, edge_attr,
           l1_W1, l1_b1, l1_W2, l1_b2, l1_root, l1_bias,
           l2_W1, l2_b1, l2_W2, l2_b2, l2_root, l2_bias,
           l3_W1, l3_b1, l3_W2, l3_b2, l3_root, l3_bias,
           l4_W1, l4_b1, l4_W2, l4_b2, l4_root, l4_bias,
           out_W, out_b):
    def pad_idx(a, fill):
        a = a.astype(jnp.int32).reshape(NW, EPW)
        a = jnp.pad(a, ((0, 0), (0, EPWP - EPW)), constant_values=fill)
        return a.reshape(NW, NCHUNK, CH)

    src = pad_idx(edge_index[0], 0)
    dst = pad_idx(edge_index[1], N)
    eat = jnp.pad(edge_attr.T.reshape(2, NW, EPW),
                  ((0, 0), (0, 0), (0, EPWP - EPW))).reshape(2, EP)
    eat = eat.reshape(2, EP // _EB, _EB // 8, 8)
    eat = eat.transpose(0, 1, 3, 2).reshape(2, EP)

    cnt2 = _sc_counts(dst)

    layers = [
        _pad_layer(l1_W1, l1_b1, l1_W2, l1_b2, l1_root, l1_bias, 1, F),
        _pad_layer(l2_W1, l2_b1, l2_W2, l2_b2, l2_root, l2_bias, F, F),
        _pad_layer(l3_W1, l3_b1, l3_W2, l3_b2, l3_root, l3_bias, F, F),
        _pad_layer(l4_W1, l4_b1, l4_W2, l4_b2, l4_root, l4_bias, F, 10),
    ]

    h = jnp.pad(x, ((0, 0), (0, F - 1)))
    out = None
    for li, (W1pT, b1pT, W2fh, W2fl, rootp, biasp) in enumerate(layers):
        xj = _sc_gather(h, src)
        msg = _tc_msg(eat, xj.reshape(EP // 8, 128), W1pT, b1pT, W2fh, W2fl)
        acc2 = _sc_scatter(msg.reshape(NW, EPWP, F), dst)
        if li < 3:
            h = _tc_update(acc2, cnt2, h, rootp, biasp)
        else:
            outWp = jnp.pad(out_W, ((0, F - 10), (0, 0)))
            out = _tc_final(acc2, cnt2, h, rootp, biasp, outWp,
                            out_b.reshape(1, 1))
    return out


# speedup vs baseline: 2.4392x; 1.1433x over previous
import functools

import jax
import jax.numpy as jnp
from jax import lax
from jax.experimental import pallas as pl
from jax.experimental.pallas import tpu as pltpu
from jax.experimental.pallas import tpu_sc as plsc

N = 10000
E = 160000
F = 16

_info = plsc.get_sparse_core_info()
NC, NS = _info.num_cores, _info.num_subcores
NW = NC * NS
EPW = E // NW
CH = 128
EPWP = 5120
EP = NW * EPWP
NCHUNK = EPWP // CH
MB = 1024
NMB = EPWP // MB
CPM = MB // CH
NP = 10240
STRIPE = NP // NS

_mesh = plsc.VectorSubcoreMesh(core_axis_name="c", subcore_axis_name="s")
_sc_params = pltpu.CompilerParams(use_tc_tiling_on_sc=False)


@functools.partial(
    pl.kernel, mesh=_mesh, compiler_params=_sc_params,
    out_type=jax.ShapeDtypeStruct((NW, EPWP, F), jnp.float32),
    scratch_types=[
        pltpu.VMEM((NCHUNK, CH), jnp.int32),
        pltpu.VMEM((EPWP, F), jnp.float32),
        pltpu.SemaphoreType.DMA,
    ],
)
def _sc_gather(table_hbm, src_hbm, out_hbm, idx_v, rows_v, sem):
    wid = lax.axis_index("s") * NC + lax.axis_index("c")
    pltpu.sync_copy(src_hbm.at[wid], idx_v)

    def body(m, carry):
        for jj in range(CPM):
            pltpu.async_copy(
                table_hbm.at[idx_v.at[m * CPM + jj]],
                rows_v.at[pl.ds((m * CPM + jj) * CH, CH)], sem)

        @pl.when(m > 0)
        def _():
            pltpu.make_async_copy(out_hbm.at[wid, pl.ds(0, MB)],
                                  rows_v.at[pl.ds(0, MB)], sem).wait()

        return carry

    lax.fori_loop(0, NMB, body, 0)
    pltpu.make_async_copy(out_hbm.at[wid, pl.ds(0, MB)],
                          rows_v.at[pl.ds(0, MB)], sem).wait()
    pltpu.sync_copy(rows_v, out_hbm.at[wid])


@functools.partial(
    pl.kernel, mesh=_mesh, compiler_params=_sc_params,
    out_type=jax.ShapeDtypeStruct((NC, NP, F), jnp.float32),
    scratch_types=[
        pltpu.VMEM((NCHUNK, CH), jnp.int32),
        pltpu.VMEM((EPWP, F), jnp.float32),
        pltpu.VMEM((STRIPE, F), jnp.float32),
        pltpu.VMEM_SHARED((NP, F), jnp.float32),
        pltpu.SemaphoreType.DMA,
    ],
)
def _sc_scatter(msg_hbm, dst_hbm, out_hbm, idx_v, msg_v, stripe_v, acc_sh, sem):
    cid = lax.axis_index("c")
    sid = lax.axis_index("s")
    wid = sid * NC + cid

    def zbody(i, carry):
        stripe_v[i, :] = jnp.zeros((F,), jnp.float32)
        return carry

    lax.fori_loop(0, STRIPE, zbody, 0)
    pltpu.sync_copy(stripe_v, acc_sh.at[pl.ds(sid * STRIPE, STRIPE)])
    pltpu.sync_copy(dst_hbm.at[wid], idx_v)
    pltpu.sync_copy(msg_hbm.at[wid], msg_v)
    plsc.subcore_barrier()

    def body(j, carry):
        pltpu.sync_copy(msg_v.at[pl.ds(j * CH, CH)],
                        acc_sh.at[idx_v.at[j]], add=True)
        return carry

    lax.fori_loop(0, NCHUNK, body, 0)
    plsc.subcore_barrier()
    pltpu.sync_copy(acc_sh.at[pl.ds(sid * STRIPE, STRIPE)], stripe_v)
    pltpu.sync_copy(stripe_v, out_hbm.at[cid, pl.ds(sid * STRIPE, STRIPE)])


@functools.partial(
    pl.kernel, mesh=_mesh, compiler_params=_sc_params,
    out_type=jax.ShapeDtypeStruct((NC, NP, F), jnp.float32),
    scratch_types=[
        pltpu.VMEM((NCHUNK, CH), jnp.int32),
        pltpu.VMEM((CH, F), jnp.float32),
        pltpu.VMEM((STRIPE, F), jnp.float32),
        pltpu.VMEM_SHARED((NP, F), jnp.float32),
    ],
)
def _sc_counts(dst_hbm, out_hbm, idx_v, ones_v, stripe_v, acc_sh):
    cid = lax.axis_index("c")
    sid = lax.axis_index("s")
    wid = sid * NC + cid

    def zbody(i, carry):
        stripe_v[i, :] = jnp.zeros((F,), jnp.float32)
        return carry

    lax.fori_loop(0, STRIPE, zbody, 0)

    def obody(i, carry):
        ones_v[i, :] = jnp.ones((F,), jnp.float32)
        return carry

    lax.fori_loop(0, CH, obody, 0)
    pltpu.sync_copy(stripe_v, acc_sh.at[pl.ds(sid * STRIPE, STRIPE)])
    pltpu.sync_copy(dst_hbm.at[wid], idx_v)
    plsc.subcore_barrier()

    def body(j, carry):
        pltpu.sync_copy(ones_v, acc_sh.at[idx_v.at[j]], add=True)
        return carry

    lax.fori_loop(0, NCHUNK, body, 0)
    plsc.subcore_barrier()
    pltpu.sync_copy(acc_sh.at[pl.ds(sid * STRIPE, STRIPE)], stripe_v)
    pltpu.sync_copy(stripe_v, out_hbm.at[cid, pl.ds(sid * STRIPE, STRIPE)])


_EB = 16384


def plt_bitcast(x):
    return lax.bitcast_convert_type(x, jnp.int32)


def plt_bitcast_f(x):
    return lax.bitcast_convert_type(x, jnp.float32)
_NB = 2000


def _msg_body(eat_ref, xj_ref, w1t_ref, b1t_ref, w3_ref, o_ref):
    xp = xj_ref[...].T
    eat = eat_ref[...]
    w1t = w1t_ref[...]
    h = jnp.maximum(
        w1t[:, 0:1] * eat[0:1, :] + w1t[:, 1:2] * eat[1:2, :]
        + b1t_ref[...], 0.0)
    w3 = w3_ref[...]
    g = _EB // 8
    cols = []
    for j in range(8):
        xjt = xp[F * j:F * (j + 1), :]
        hj = h[:, g * j:g * (j + 1)]
        xb = plt_bitcast(xjt)
        xh = plt_bitcast_f(xb & jnp.int32(-65536))
        xl = xjt - xh
        xcat = jnp.concatenate(
            [xh.astype(jnp.bfloat16), xl.astype(jnp.bfloat16),
             xh.astype(jnp.bfloat16)], axis=0)
        t = jnp.dot(w3, xcat, preferred_element_type=jnp.float32)
        acc = hj[10:11, :] * t[160:176, :]
        for k in range(10):
            acc = acc + hj[k:k + 1, :] * t[k * F:(k + 1) * F, :]
        cols.append(acc.T)
    o_ref[...] = jnp.concatenate(cols, axis=1)


def _tc_msg(eat, xj, W1pT, b1pT, W3):
    return pl.pallas_call(
        _msg_body,
        grid=(EP // _EB,),
        in_specs=[
            pl.BlockSpec((2, _EB), lambda i: (0, i)),
            pl.BlockSpec((_EB // 8, 128), lambda i: (i, 0)),
            pl.BlockSpec((F, 2), lambda i: (0, 0)),
            pl.BlockSpec((F, 1), lambda i: (0, 0)),
            pl.BlockSpec((11 * F, 48), lambda i: (0, 0)),
        ],
        out_specs=pl.BlockSpec((_EB // 8, 128), lambda i: (i, 0)),
        out_shape=jax.ShapeDtypeStruct((EP // 8, 128), jnp.float32),
    )(eat, xj, W1pT, b1pT, W3)


def _update_body(acc_ref, cnt_ref, x_ref, root_ref, bias_ref, o_ref):
    s = acc_ref[0] + acc_ref[1]
    c = cnt_ref[0] + cnt_ref[1]
    mean = s / jnp.maximum(c, 1.0)
    o_ref[...] = jnp.maximum(
        mean + jnp.dot(x_ref[...], root_ref[...],
                       preferred_element_type=jnp.float32,
                 precision=lax.Precision.HIGHEST)
        + bias_ref[...], 0.0)


def _tc_update(acc2, cnt2, x, rootp, biasp):
    return pl.pallas_call(
        _update_body,
        grid=(N // _NB,),
        in_specs=[
            pl.BlockSpec((2, _NB, F), lambda i: (0, i, 0)),
            pl.BlockSpec((2, _NB, F), lambda i: (0, i, 0)),
            pl.BlockSpec((_NB, F), lambda i: (i, 0)),
            pl.BlockSpec((F, F), lambda i: (0, 0)),
            pl.BlockSpec((1, F), lambda i: (0, 0)),
        ],
        out_specs=pl.BlockSpec((_NB, F), lambda i: (i, 0)),
        out_shape=jax.ShapeDtypeStruct((N, F), jnp.float32),
    )(acc2, cnt2, x, rootp, biasp)


def _final_body(acc_ref, cnt_ref, x_ref, root_ref, bias_ref, ow_ref, ob_ref,
                o_ref):
    s = acc_ref[0] + acc_ref[1]
    c = cnt_ref[0] + cnt_ref[1]
    mean = s / jnp.maximum(c, 1.0)
    h = jnp.maximum(
        mean + jnp.dot(x_ref[...], root_ref[...],
                       preferred_element_type=jnp.float32,
                 precision=lax.Precision.HIGHEST)
        + bias_ref[...], 0.0)
    o_ref[...] = jnp.dot(h, ow_ref[...],
                         preferred_element_type=jnp.float32,
                 precision=lax.Precision.HIGHEST) + ob_ref[...]


def _tc_final(acc2, cnt2, x, rootp, biasp, outWp, out_b):
    return pl.pallas_call(
        _final_body,
        grid=(N // _NB,),
        in_specs=[
            pl.BlockSpec((2, _NB, F), lambda i: (0, i, 0)),
            pl.BlockSpec((2, _NB, F), lambda i: (0, i, 0)),
            pl.BlockSpec((_NB, F), lambda i: (i, 0)),
            pl.BlockSpec((F, F), lambda i: (0, 0)),
            pl.BlockSpec((1, F), lambda i: (0, 0)),
            pl.BlockSpec((F, 1), lambda i: (0, 0)),
            pl.BlockSpec((1, 1), lambda i: (0, 0)),
        ],
        out_specs=pl.BlockSpec((_NB, 1), lambda i: (i, 0)),
        out_shape=jax.ShapeDtypeStruct((N, 1), jnp.float32),
    )(acc2, cnt2, x, rootp, biasp, outWp, out_b)


def _pad_layer(W1, b1, W2, b2, root, bias, cin, cout):
    W1p = jnp.pad(W1, ((0, 0), (0, F - 10)))
    b1p = jnp.pad(b1, (0, F - 10)).at[10].set(1.0).reshape(1, F)
    W2r = jnp.pad(W2.reshape(10, cin, cout),
                  ((0, 0), (0, F - cin), (0, F - cout)))
    B2r = jnp.pad(b2.reshape(cin, cout), ((0, F - cin), (0, F - cout)))
    W2s = jnp.concatenate([W2r, B2r[None]], axis=0)
    W2f = jnp.transpose(W2s, (0, 2, 1)).reshape(11 * F, F)
    W2fi = jax.lax.bitcast_convert_type(W2f, jnp.int32)
    W2fh = jax.lax.bitcast_convert_type(W2fi & jnp.int32(-65536), jnp.float32)
    W2fl = (W2f - W2fh).astype(jnp.bfloat16)
    W2fh = W2fh.astype(jnp.bfloat16)
    W3 = jnp.concatenate([W2fh, W2fh, W2fl], axis=1)
    rootp = jnp.pad(root, ((0, F - cin), (0, F - cout)))
    biasp = jnp.pad(bias, (0, F - cout)).reshape(1, F)
    return W1p.T, b1p.reshape(F, 1), W3, rootp, biasp


def kernel(x, edge_index, edge_attr,
           l1_W1, l1_b1, l1_W2, l1_b2, l1_root, l1_bias,
           l2_W1, l2_b1, l2_W2, l2_b2, l2_root, l2_bias,
           l3_W1, l3_b1, l3_W2, l3_b2, l3_root, l3_bias,
           l4_W1, l4_b1, l4_W2, l4_b2, l4_root, l4_bias,
           out_W, out_b):
    def pad_idx(a, fill):
        a = a.astype(jnp.int32).reshape(NW, EPW)
        a = jnp.pad(a, ((0, 0), (0, EPWP - EPW)), constant_values=fill)
        return a.reshape(NW, NCHUNK, CH)

    src = pad_idx(edge_index[0], 0)
    dst = pad_idx(edge_index[1], N)
    eat = jnp.pad(edge_attr.T.reshape(2, NW, EPW),
                  ((0, 0), (0, 0), (0, EPWP - EPW))).reshape(2, EP)
    eat = eat.reshape(2, EP // _EB, _EB // 8, 8)
    eat = eat.transpose(0, 1, 3, 2).reshape(2, EP)

    cnt2 = _sc_counts(dst)

    layers = [
        _pad_layer(l1_W1, l1_b1, l1_W2, l1_b2, l1_root, l1_bias, 1, F),
        _pad_layer(l2_W1, l2_b1, l2_W2, l2_b2, l2_root, l2_bias, F, F),
        _pad_layer(l3_W1, l3_b1, l3_W2, l3_b2, l3_root, l3_bias, F, F),
        _pad_layer(l4_W1, l4_b1, l4_W2, l4_b2, l4_root, l4_bias, F, 10),
    ]

    h = jnp.pad(x, ((0, 0), (0, F - 1)))
    out = None
    for li, (W1pT, b1pT, W3, rootp, biasp) in enumerate(layers):
        xj = _sc_gather(h, src)
        msg = _tc_msg(eat, xj.reshape(EP // 8, 128), W1pT, b1pT, W3)
        acc2 = _sc_scatter(msg.reshape(NW, EPWP, F), dst)
        if li < 3:
            h = _tc_update(acc2, cnt2, h, rootp, biasp)
        else:
            outWp = jnp.pad(out_W, ((0, F - 10), (0, 0)))
            out = _tc_final(acc2, cnt2, h, rootp, biasp, outWp,
                            out_b.reshape(1, 1))
    return out
